# Initial kernel scaffold; baseline (speedup 1.0000x reference)
import jax
import jax.numpy as jnp
from jax.experimental import pallas as pl


def kernel(image_features, vertices, edge_index, W_self0, W_nbr0, b0, W_self1, W_nbr1, b1, W_self2, W_nbr2, b2):
    raise NotImplementedError("write your pallas kernel here")


# speedup vs baseline: 5.6370x; 5.6370x over previous
import functools

import jax
import jax.numpy as jnp
from jax import lax
from jax.experimental import pallas as pl
from jax.experimental.pallas import tpu as pltpu
from jax.experimental.pallas import tpu_sc as plsc

V = 50000
E = 800000
GRID = 64
NC = 2
NS = 16
NW = NC * NS

VACC = 51200
RPS = VACC // NS
EPAD = 819200
EBLK = EPAD // NW // 128
CBLK = 104
CIDX_N = NW * CBLK * 128
CPAD = CIDX_N // 8

_LEAK = 0.3
_SCALE = 0.1


def _sc_mesh():
    return plsc.VectorSubcoreMesh(
        core_axis_name="c", subcore_axis_name="s", num_cores=NC, num_subcores=NS
    )


def _corner_gather(tbl, idx2):
    @functools.partial(
        pl.kernel,
        out_type=jax.ShapeDtypeStruct((CIDX_N, 32), jnp.float32),
        mesh=_sc_mesh(),
        compiler_params=pltpu.CompilerParams(use_tc_tiling_on_sc=False),
        scratch_types=[
            pltpu.VMEM((CBLK, 128), jnp.int32),
            pltpu.VMEM((128, 32), jnp.float32),
            pltpu.SemaphoreType.DMA,
        ],
    )
    def k(tbl_h, idx_h, out_h, idx_v, rows_v, sem):
        c = lax.axis_index("c")
        s = lax.axis_index("s")
        wid = s * NC + c
        pltpu.sync_copy(idx_h.at[pl.ds(wid * CBLK, CBLK)], idx_v)

        def body(j, _):
            pltpu.async_copy(tbl_h.at[idx_v.at[j]], rows_v, sem).wait()
            pltpu.sync_copy(rows_v, out_h.at[pl.ds(wid * CBLK * 128 + j * 128, 128)])
            return 0

        lax.fori_loop(0, CBLK, body, 0)

    return k(tbl, idx2)


def _make_edge_pass(D):
    @functools.partial(
        pl.kernel,
        out_type=jax.ShapeDtypeStruct((NC, VACC, D), jnp.float32),
        mesh=_sc_mesh(),
        compiler_params=pltpu.CompilerParams(use_tc_tiling_on_sc=False),
        scratch_types=[
            pltpu.VMEM((EBLK, 128), jnp.int32),
            pltpu.VMEM((EBLK, 128), jnp.int32),
            pltpu.VMEM((128, D), jnp.float32),
            pltpu.VMEM_SHARED((VACC, D), jnp.float32),
            pltpu.SemaphoreType.DMA,
        ],
    )
    def k(tbl_h, src_h, dst_h, out_h, src_v, dst_v, rows_v, acc, sem):
        c = lax.axis_index("c")
        s = lax.axis_index("s")
        wid = s * NC + c

        def zrow(i, _):
            for kk in range(D // 16):
                rows_v[i, pl.ds(kk * 16, 16)] = jnp.zeros((16,), jnp.float32)
            return 0

        lax.fori_loop(0, 128, zrow, 0)

        def zacc(i, _):
            pltpu.sync_copy(rows_v, acc.at[pl.ds(s * RPS + i * 128, 128)])
            return 0

        lax.fori_loop(0, RPS // 128, zacc, 0)

        pltpu.sync_copy(src_h.at[pl.ds(wid * EBLK, EBLK)], src_v)
        pltpu.sync_copy(dst_h.at[pl.ds(wid * EBLK, EBLK)], dst_v)
        plsc.subcore_barrier()

        def body(j, _):
            pltpu.async_copy(tbl_h.at[src_v.at[j]], rows_v, sem).wait()
            pltpu.sync_copy(rows_v, acc.at[dst_v.at[j]], add=True)
            return 0

        lax.fori_loop(0, EBLK, body, 0)
        plsc.subcore_barrier()
        pltpu.sync_copy(
            acc.at[pl.ds(s * RPS, RPS)], out_h.at[c, pl.ds(s * RPS, RPS)]
        )

    return k


def _prep(vertices):
    def body(v_ref, cidx_ref, cw_ref):
        v = v_ref[...]
        vc = jnp.clip(v, 0.0, GRID - 1.0 - 1e-4)
        v0 = jnp.floor(vc)
        f = vc - v0
        i0 = v0.astype(jnp.int32)
        i1 = jnp.minimum(i0 + 1, GRID - 1)
        x0, y0, z0 = i0[:, 0:1], i0[:, 1:2], i0[:, 2:3]
        x1, y1, z1 = i1[:, 0:1], i1[:, 1:2], i1[:, 2:3]
        fx, fy, fz = f[:, 0:1], f[:, 1:2], f[:, 2:3]
        gx, gy, gz = 1.0 - fx, 1.0 - fy, 1.0 - fz
        flat = lambda a, b, cc: (a * GRID + b) * GRID + cc
        cidx_ref[...] = jnp.concatenate(
            [flat(x0, y0, z0), flat(x1, y0, z0), flat(x0, y1, z0),
             flat(x0, y0, z1), flat(x1, y1, z0), flat(x1, y0, z1),
             flat(x0, y1, z1), flat(x1, y1, z1)], axis=1)
        cw_ref[...] = jnp.concatenate(
            [gx * gy * gz, fx * gy * gz, gx * fy * gz, gx * gy * fz,
             fx * fy * gz, fx * gy * fz, gx * fy * fz, fx * fy * fz], axis=1)

    return pl.pallas_call(
        body,
        grid=(_NBLK_TC,),
        in_specs=[pl.BlockSpec((_VB, 3), lambda i: (i, 0))],
        out_specs=[
            pl.BlockSpec((_VB, 8), lambda i: (i, 0)),
            pl.BlockSpec((_VB, 8), lambda i: (i, 0)),
        ],
        out_shape=(
            jax.ShapeDtypeStruct((V, 8), jnp.int32),
            jax.ShapeDtypeStruct((V, 8), jnp.float32),
        ),
    )(vertices)


_VB = 2000
_NBLK_TC = V // _VB


def _chunk_specs(n, width):
    return [pl.BlockSpec((_VB, width), lambda i: (i, 0)) for _ in range(n)]


def _part_specs(n):
    return [pl.BlockSpec((NC, _VB, 16), lambda i: (0, i, 0)) for _ in range(n)]


def _build_h0(corners, cw, vertices):
    def body(cor_ref, cw_ref, v_ref, h0_ref, h1_ref, h2_ref):
        cor = cor_ref[...]
        cw_b = cw_ref[...]
        feats = cor[0] * cw_b[:, 0:1]
        for cc in range(1, 8):
            feats = feats + cor[cc] * cw_b[:, cc:cc + 1]
        pos = v_ref[...] * (1.0 / (GRID - 1.0))
        nrow = feats.shape[0]
        h0_ref[...] = jnp.concatenate([pos, feats[:, :13]], axis=1)
        h1_ref[...] = feats[:, 13:29]
        h2_ref[...] = jnp.concatenate(
            [feats[:, 29:32],
             jnp.ones((nrow, 1), jnp.float32),
             jnp.zeros((nrow, 12), jnp.float32)], axis=1)

    return pl.pallas_call(
        body,
        grid=(_NBLK_TC,),
        in_specs=[
            pl.BlockSpec((8, _VB, 32), lambda i: (0, i, 0)),
            pl.BlockSpec((_VB, 8), lambda i: (i, 0)),
            pl.BlockSpec((_VB, 3), lambda i: (i, 0)),
        ],
        out_specs=_chunk_specs(3, 16),
        out_shape=tuple(
            jax.ShapeDtypeStruct((V, 16), jnp.float32) for _ in range(3)),
    )(corners, cw, vertices)


def _dense0(h0c, p0c, ws, wn, b):
    def body(ha_ref, hb_ref, hc_ref, pa_ref, pb_ref, pc_ref,
             ws_ref, wn_ref, b_ref, o0_ref, o1_ref, o2_ref, o3_ref, cnt_ref):
        h0 = jnp.concatenate(
            [ha_ref[...], hb_ref[...], hc_ref[...]], axis=1)
        s = jnp.concatenate(
            [pa_ref[0] + pa_ref[1], pb_ref[0] + pb_ref[1],
             pc_ref[0] + pc_ref[1]], axis=1)
        cr = s[:, 35:36]
        inv = 1.0 / jnp.maximum(cr, 1.0)
        r = cr * inv
        t = s * inv - r * h0
        z = (jnp.dot(h0, ws_ref[...], preferred_element_type=jnp.float32)
             + jnp.dot(t, wn_ref[...], preferred_element_type=jnp.float32)
             + b_ref[...])
        h1 = jnp.where(z >= 0, z, _LEAK * z)
        o0_ref[...] = h1[:, 0:16]
        o1_ref[...] = h1[:, 16:32]
        o2_ref[...] = h1[:, 32:48]
        o3_ref[...] = h1[:, 48:64]
        cnt_ref[...] = jnp.concatenate(
            [inv, r, jnp.zeros((inv.shape[0], 6), jnp.float32)], axis=1)

    return pl.pallas_call(
        body,
        grid=(_NBLK_TC,),
        in_specs=_chunk_specs(3, 16) + _part_specs(3) + [
            pl.BlockSpec((48, 64), lambda i: (0, 0)),
            pl.BlockSpec((48, 64), lambda i: (0, 0)),
            pl.BlockSpec((1, 64), lambda i: (0, 0)),
        ],
        out_specs=_chunk_specs(4, 16) + [
            pl.BlockSpec((_VB, 8), lambda i: (i, 0))],
        out_shape=tuple(
            jax.ShapeDtypeStruct((V, 16), jnp.float32) for _ in range(4)
        ) + (jax.ShapeDtypeStruct((V, 8), jnp.float32),),
    )(*h0c, *p0c, ws, wn, b)


def _dense1(h1c, p1c, cnt, ws1, wn1, b1, ws2, wn2):
    def body(h0_ref, h1_ref, h2_ref, h3_ref, p0_ref, p1_ref, p2_ref, p3_ref,
             cnt_ref, ws1_ref, wn1_ref, b1_ref, ws2_ref, wn2_ref,
             m_ref, u_ref):
        h1 = jnp.concatenate(
            [h0_ref[...], h1_ref[...], h2_ref[...], h3_ref[...]], axis=1)
        s = jnp.concatenate(
            [p0_ref[0] + p0_ref[1], p1_ref[0] + p1_ref[1],
             p2_ref[0] + p2_ref[1], p3_ref[0] + p3_ref[1]], axis=1)
        inv = cnt_ref[:, 0:1]
        r = cnt_ref[:, 1:2]
        t = s * inv - r * h1
        z = (jnp.dot(h1, ws1_ref[...], preferred_element_type=jnp.float32)
             + jnp.dot(t, wn1_ref[...], preferred_element_type=jnp.float32)
             + b1_ref[...])
        h2 = jnp.where(z >= 0, z, _LEAK * z)
        m2 = jnp.dot(h2, wn2_ref[...], preferred_element_type=jnp.float32)
        u2 = jnp.dot(h2, ws2_ref[...], preferred_element_type=jnp.float32)
        nrow = h2.shape[0]
        m_ref[...] = jnp.concatenate(
            [m2, jnp.zeros((nrow, 13), jnp.float32)], axis=1)
        u_ref[...] = jnp.concatenate(
            [u2, jnp.zeros((nrow, 5), jnp.float32)], axis=1)

    return pl.pallas_call(
        body,
        grid=(_NBLK_TC,),
        in_specs=_chunk_specs(4, 16) + _part_specs(4) + [
            pl.BlockSpec((_VB, 8), lambda i: (i, 0)),
            pl.BlockSpec((64, 64), lambda i: (0, 0)),
            pl.BlockSpec((64, 64), lambda i: (0, 0)),
            pl.BlockSpec((1, 64), lambda i: (0, 0)),
            pl.BlockSpec((64, 3), lambda i: (0, 0)),
            pl.BlockSpec((64, 3), lambda i: (0, 0)),
        ],
        out_specs=[
            pl.BlockSpec((_VB, 16), lambda i: (i, 0)),
            pl.BlockSpec((_VB, 8), lambda i: (i, 0)),
        ],
        out_shape=(
            jax.ShapeDtypeStruct((V, 16), jnp.float32),
            jax.ShapeDtypeStruct((V, 8), jnp.float32),
        ),
    )(*h1c, *p1c, cnt, ws1, wn1, b1, ws2, wn2)


def _final(vertices, u2, m2p, p2, cnt, b2p):
    def body(v_ref, u_ref, m_ref, p_ref, cnt_ref, b_ref, o_ref):
        s2 = p_ref[0] + p_ref[1]
        inv = cnt_ref[:, 0:1]
        r = cnt_ref[:, 1:2]
        agg = s2[:, 0:3] * inv - r * m_ref[:, 0:3]
        d = u_ref[:, 0:3] + agg + b_ref[:, 0:3]
        o_ref[...] = v_ref[...] + _SCALE * d

    return pl.pallas_call(
        body,
        grid=(_NBLK_TC,),
        in_specs=[
            pl.BlockSpec((_VB, 3), lambda i: (i, 0)),
            pl.BlockSpec((_VB, 8), lambda i: (i, 0)),
            pl.BlockSpec((_VB, 16), lambda i: (i, 0)),
            pl.BlockSpec((NC, _VB, 16), lambda i: (0, i, 0)),
            pl.BlockSpec((_VB, 8), lambda i: (i, 0)),
            pl.BlockSpec((1, 8), lambda i: (0, 0)),
        ],
        out_specs=pl.BlockSpec((_VB, 3), lambda i: (i, 0)),
        out_shape=jax.ShapeDtypeStruct((V, 3), jnp.float32),
    )(vertices, u2, m2p, p2, cnt, b2p)


def kernel(image_features, vertices, edge_index, W_self0, W_nbr0, b0,
           W_self1, W_nbr1, b1, W_self2, W_nbr2, b2):
    tbl_img = image_features.reshape(32, GRID * GRID * GRID).T

    src = edge_index[0]
    dst = edge_index[1]
    npad = EPAD - E
    ar = jnp.arange(npad, dtype=jnp.int32)
    src2 = jnp.concatenate([src, (ar * 37) % V]).reshape(EPAD // 128, 128)
    dst2 = jnp.concatenate([dst, V + ar % (VACC - V)]).reshape(EPAD // 128, 128)

    ws0p = jnp.pad(W_self0, ((0, 13), (0, 0)))
    wn0p = jnp.pad(W_nbr0, ((0, 13), (0, 0)))
    b0r = b0.reshape(1, 64)
    b1r = b1.reshape(1, 64)
    b2p = jnp.pad(b2, (0, 5)).reshape(1, 8)

    cidx, cw = _prep(vertices)
    cidx8 = jnp.pad(cidx.T, ((0, 0), (0, CPAD - V))).reshape(CIDX_N // 128, 128)
    corners = _corner_gather(tbl_img, cidx8).reshape(8, CPAD, 32)

    h0c = _build_h0(corners, cw, vertices)

    ep16 = _make_edge_pass(16)

    p0c = tuple(ep16(h, src2, dst2) for h in h0c)
    h1_0, h1_1, h1_2, h1_3, cnt = _dense0(h0c, p0c, ws0p, wn0p, b0r)

    h1c = (h1_0, h1_1, h1_2, h1_3)
    p1c = tuple(ep16(h, src2, dst2) for h in h1c)
    m2p, u2 = _dense1(h1c, p1c, cnt, W_self1, W_nbr1, b1r, W_self2, W_nbr2)

    p2 = ep16(m2p, src2, dst2)
    return _final(vertices, u2, m2p, p2, cnt, b2p)


# speedup vs baseline: 9.4915x; 1.6838x over previous
import functools

import jax
import jax.numpy as jnp
from jax import lax
from jax.experimental import pallas as pl
from jax.experimental.pallas import tpu as pltpu
from jax.experimental.pallas import tpu_sc as plsc

V = 50000
E = 800000
GRID = 64
NC = 2
NS = 16
NW = NC * NS

VACC = 51200
RPS = VACC // NS
EPAD = 819200
EBLK = EPAD // NW // 128
CBLK = 104
CIDX_N = NW * CBLK * 128
CPAD = CIDX_N // 8

_LEAK = 0.3
_SCALE = 0.1
_NBUF = 8


def _sc_mesh():
    return plsc.VectorSubcoreMesh(
        core_axis_name="c", subcore_axis_name="s", num_cores=NC, num_subcores=NS
    )


def _corner_gather(tbl, idx2):
    @functools.partial(
        pl.kernel,
        out_type=jax.ShapeDtypeStruct((CIDX_N, 32), jnp.float32),
        mesh=_sc_mesh(),
        compiler_params=pltpu.CompilerParams(use_tc_tiling_on_sc=False),
        scratch_types=[
            pltpu.VMEM((CBLK, 128), jnp.int32),
            pltpu.VMEM((_NBUF, 128, 32), jnp.float32),
        ] + [pltpu.SemaphoreType.DMA] * _NBUF,
    )
    def k(tbl_h, idx_h, out_h, idx_v, rows_v, *sems):
        c = lax.axis_index("c")
        s = lax.axis_index("s")
        wid = s * NC + c
        pltpu.sync_copy(idx_h.at[pl.ds(wid * CBLK, CBLK)], idx_v)

        for b in range(_NBUF):
            pltpu.async_copy(tbl_h.at[idx_v.at[b]], rows_v.at[b], sems[b])

        def group(g, _):
            for b in range(_NBUF):
                j = g * _NBUF + b
                pltpu.make_async_copy(
                    tbl_h.at[idx_v.at[j]], rows_v.at[b], sems[b]).wait()
                pltpu.sync_copy(
                    rows_v.at[b],
                    out_h.at[pl.ds(wid * CBLK * 128 + j * 128, 128)])
                jn = j + _NBUF

                @pl.when(jn < CBLK)
                def _():
                    pltpu.async_copy(
                        tbl_h.at[idx_v.at[jn]], rows_v.at[b], sems[b])
            return 0

        lax.fori_loop(0, CBLK // _NBUF, group, 0)

    return k(tbl, idx2)


def _make_edge_pass(D):
    @functools.partial(
        pl.kernel,
        out_type=jax.ShapeDtypeStruct((NC, VACC, D), jnp.float32),
        mesh=_sc_mesh(),
        compiler_params=pltpu.CompilerParams(use_tc_tiling_on_sc=False),
        scratch_types=[
            pltpu.VMEM((EBLK, 128), jnp.int32),
            pltpu.VMEM((EBLK, 128), jnp.int32),
            pltpu.VMEM((_NBUF, 128, D), jnp.float32),
            pltpu.VMEM_SHARED((VACC, D), jnp.float32),
        ] + [pltpu.SemaphoreType.DMA] * _NBUF,
    )
    def k(tbl_h, src_h, dst_h, out_h, src_v, dst_v, rows_v, acc, *sems):
        c = lax.axis_index("c")
        s = lax.axis_index("s")
        wid = s * NC + c

        def zrow(i, _):
            for kk in range(D // 16):
                rows_v[0, i, pl.ds(kk * 16, 16)] = jnp.zeros((16,), jnp.float32)
            return 0

        lax.fori_loop(0, 128, zrow, 0)

        def zacc(i, _):
            pltpu.sync_copy(rows_v.at[0], acc.at[pl.ds(s * RPS + i * 128, 128)])
            return 0

        lax.fori_loop(0, RPS // 128, zacc, 0)

        pltpu.sync_copy(src_h.at[pl.ds(wid * EBLK, EBLK)], src_v)
        pltpu.sync_copy(dst_h.at[pl.ds(wid * EBLK, EBLK)], dst_v)
        plsc.subcore_barrier()

        for b in range(_NBUF):
            pltpu.async_copy(tbl_h.at[src_v.at[b]], rows_v.at[b], sems[b])

        def group(g, _):
            for b in range(_NBUF):
                j = g * _NBUF + b
                pltpu.make_async_copy(
                    tbl_h.at[src_v.at[j]], rows_v.at[b], sems[b]).wait()
                pltpu.sync_copy(rows_v.at[b], acc.at[dst_v.at[j]], add=True)
                jn = j + _NBUF

                @pl.when(jn < EBLK)
                def _():
                    pltpu.async_copy(
                        tbl_h.at[src_v.at[jn]], rows_v.at[b], sems[b])
            return 0

        lax.fori_loop(0, EBLK // _NBUF, group, 0)
        plsc.subcore_barrier()
        pltpu.sync_copy(
            acc.at[pl.ds(s * RPS, RPS)], out_h.at[c, pl.ds(s * RPS, RPS)]
        )

    return k


def _prep(vertices):
    def body(v_ref, cidx_ref, cw_ref):
        v = v_ref[...]
        vc = jnp.clip(v, 0.0, GRID - 1.0 - 1e-4)
        v0 = jnp.floor(vc)
        f = vc - v0
        i0 = v0.astype(jnp.int32)
        i1 = jnp.minimum(i0 + 1, GRID - 1)
        x0, y0, z0 = i0[:, 0:1], i0[:, 1:2], i0[:, 2:3]
        x1, y1, z1 = i1[:, 0:1], i1[:, 1:2], i1[:, 2:3]
        fx, fy, fz = f[:, 0:1], f[:, 1:2], f[:, 2:3]
        gx, gy, gz = 1.0 - fx, 1.0 - fy, 1.0 - fz
        flat = lambda a, b, cc: (a * GRID + b) * GRID + cc
        cidx_ref[...] = jnp.concatenate(
            [flat(x0, y0, z0), flat(x1, y0, z0), flat(x0, y1, z0),
             flat(x0, y0, z1), flat(x1, y1, z0), flat(x1, y0, z1),
             flat(x0, y1, z1), flat(x1, y1, z1)], axis=1)
        cw_ref[...] = jnp.concatenate(
            [gx * gy * gz, fx * gy * gz, gx * fy * gz, gx * gy * fz,
             fx * fy * gz, fx * gy * fz, gx * fy * fz, fx * fy * fz], axis=1)

    return pl.pallas_call(
        body,
        grid=(_NBLK_TC,),
        in_specs=[pl.BlockSpec((_VB, 3), lambda i: (i, 0))],
        out_specs=[
            pl.BlockSpec((_VB, 8), lambda i: (i, 0)),
            pl.BlockSpec((_VB, 8), lambda i: (i, 0)),
        ],
        out_shape=(
            jax.ShapeDtypeStruct((V, 8), jnp.int32),
            jax.ShapeDtypeStruct((V, 8), jnp.float32),
        ),
    )(vertices)


_VB = 2000
_NBLK_TC = V // _VB


def _chunk_specs(n, width):
    return [pl.BlockSpec((_VB, width), lambda i: (i, 0)) for _ in range(n)]


def _part_specs(n):
    return [pl.BlockSpec((NC, _VB, 16), lambda i: (0, i, 0)) for _ in range(n)]


def _build_h0(corners, cw, vertices):
    def body(cor_ref, cw_ref, v_ref, h0_ref, h1_ref, h2_ref):
        cor = cor_ref[...]
        cw_b = cw_ref[...]
        feats = cor[0] * cw_b[:, 0:1]
        for cc in range(1, 8):
            feats = feats + cor[cc] * cw_b[:, cc:cc + 1]
        pos = v_ref[...] * (1.0 / (GRID - 1.0))
        nrow = feats.shape[0]
        h0_ref[...] = jnp.concatenate([pos, feats[:, :13]], axis=1)
        h1_ref[...] = feats[:, 13:29]
        h2_ref[...] = jnp.concatenate(
            [feats[:, 29:32],
             jnp.ones((nrow, 1), jnp.float32),
             jnp.zeros((nrow, 12), jnp.float32)], axis=1)

    return pl.pallas_call(
        body,
        grid=(_NBLK_TC,),
        in_specs=[
            pl.BlockSpec((8, _VB, 32), lambda i: (0, i, 0)),
            pl.BlockSpec((_VB, 8), lambda i: (i, 0)),
            pl.BlockSpec((_VB, 3), lambda i: (i, 0)),
        ],
        out_specs=_chunk_specs(3, 16),
        out_shape=tuple(
            jax.ShapeDtypeStruct((V, 16), jnp.float32) for _ in range(3)),
    )(corners, cw, vertices)


def _dense0(h0c, p0c, ws, wn, b):
    def body(ha_ref, hb_ref, hc_ref, pa_ref, pb_ref, pc_ref,
             ws_ref, wn_ref, b_ref, o0_ref, o1_ref, o2_ref, o3_ref, cnt_ref):
        h0 = jnp.concatenate(
            [ha_ref[...], hb_ref[...], hc_ref[...]], axis=1)
        s = jnp.concatenate(
            [pa_ref[0] + pa_ref[1], pb_ref[0] + pb_ref[1],
             pc_ref[0] + pc_ref[1]], axis=1)
        cr = s[:, 35:36]
        inv = 1.0 / jnp.maximum(cr, 1.0)
        r = cr * inv
        t = s * inv - r * h0
        z = (jnp.dot(h0, ws_ref[...], preferred_element_type=jnp.float32)
             + jnp.dot(t, wn_ref[...], preferred_element_type=jnp.float32)
             + b_ref[...])
        h1 = jnp.where(z >= 0, z, _LEAK * z)
        o0_ref[...] = h1[:, 0:16]
        o1_ref[...] = h1[:, 16:32]
        o2_ref[...] = h1[:, 32:48]
        o3_ref[...] = h1[:, 48:64]
        cnt_ref[...] = jnp.concatenate(
            [inv, r, jnp.zeros((inv.shape[0], 6), jnp.float32)], axis=1)

    return pl.pallas_call(
        body,
        grid=(_NBLK_TC,),
        in_specs=_chunk_specs(3, 16) + _part_specs(3) + [
            pl.BlockSpec((48, 64), lambda i: (0, 0)),
            pl.BlockSpec((48, 64), lambda i: (0, 0)),
            pl.BlockSpec((1, 64), lambda i: (0, 0)),
        ],
        out_specs=_chunk_specs(4, 16) + [
            pl.BlockSpec((_VB, 8), lambda i: (i, 0))],
        out_shape=tuple(
            jax.ShapeDtypeStruct((V, 16), jnp.float32) for _ in range(4)
        ) + (jax.ShapeDtypeStruct((V, 8), jnp.float32),),
    )(*h0c, *p0c, ws, wn, b)


def _dense1(h1c, p1c, cnt, ws1, wn1, b1, ws2, wn2):
    def body(h0_ref, h1_ref, h2_ref, h3_ref, p0_ref, p1_ref, p2_ref, p3_ref,
             cnt_ref, ws1_ref, wn1_ref, b1_ref, ws2_ref, wn2_ref,
             m_ref, u_ref):
        h1 = jnp.concatenate(
            [h0_ref[...], h1_ref[...], h2_ref[...], h3_ref[...]], axis=1)
        s = jnp.concatenate(
            [p0_ref[0] + p0_ref[1], p1_ref[0] + p1_ref[1],
             p2_ref[0] + p2_ref[1], p3_ref[0] + p3_ref[1]], axis=1)
        inv = cnt_ref[:, 0:1]
        r = cnt_ref[:, 1:2]
        t = s * inv - r * h1
        z = (jnp.dot(h1, ws1_ref[...], preferred_element_type=jnp.float32)
             + jnp.dot(t, wn1_ref[...], preferred_element_type=jnp.float32)
             + b1_ref[...])
        h2 = jnp.where(z >= 0, z, _LEAK * z)
        m2 = jnp.dot(h2, wn2_ref[...], preferred_element_type=jnp.float32)
        u2 = jnp.dot(h2, ws2_ref[...], preferred_element_type=jnp.float32)
        nrow = h2.shape[0]
        m_ref[...] = jnp.concatenate(
            [m2, jnp.zeros((nrow, 13), jnp.float32)], axis=1)
        u_ref[...] = jnp.concatenate(
            [u2, jnp.zeros((nrow, 5), jnp.float32)], axis=1)

    return pl.pallas_call(
        body,
        grid=(_NBLK_TC,),
        in_specs=_chunk_specs(4, 16) + _part_specs(4) + [
            pl.BlockSpec((_VB, 8), lambda i: (i, 0)),
            pl.BlockSpec((64, 64), lambda i: (0, 0)),
            pl.BlockSpec((64, 64), lambda i: (0, 0)),
            pl.BlockSpec((1, 64), lambda i: (0, 0)),
            pl.BlockSpec((64, 3), lambda i: (0, 0)),
            pl.BlockSpec((64, 3), lambda i: (0, 0)),
        ],
        out_specs=[
            pl.BlockSpec((_VB, 16), lambda i: (i, 0)),
            pl.BlockSpec((_VB, 8), lambda i: (i, 0)),
        ],
        out_shape=(
            jax.ShapeDtypeStruct((V, 16), jnp.float32),
            jax.ShapeDtypeStruct((V, 8), jnp.float32),
        ),
    )(*h1c, *p1c, cnt, ws1, wn1, b1, ws2, wn2)


def _final(vertices, u2, m2p, p2, cnt, b2p):
    def body(v_ref, u_ref, m_ref, p_ref, cnt_ref, b_ref, o_ref):
        s2 = p_ref[0] + p_ref[1]
        inv = cnt_ref[:, 0:1]
        r = cnt_ref[:, 1:2]
        agg = s2[:, 0:3] * inv - r * m_ref[:, 0:3]
        d = u_ref[:, 0:3] + agg + b_ref[:, 0:3]
        o_ref[...] = v_ref[...] + _SCALE * d

    return pl.pallas_call(
        body,
        grid=(_NBLK_TC,),
        in_specs=[
            pl.BlockSpec((_VB, 3), lambda i: (i, 0)),
            pl.BlockSpec((_VB, 8), lambda i: (i, 0)),
            pl.BlockSpec((_VB, 16), lambda i: (i, 0)),
            pl.BlockSpec((NC, _VB, 16), lambda i: (0, i, 0)),
            pl.BlockSpec((_VB, 8), lambda i: (i, 0)),
            pl.BlockSpec((1, 8), lambda i: (0, 0)),
        ],
        out_specs=pl.BlockSpec((_VB, 3), lambda i: (i, 0)),
        out_shape=jax.ShapeDtypeStruct((V, 3), jnp.float32),
    )(vertices, u2, m2p, p2, cnt, b2p)


def kernel(image_features, vertices, edge_index, W_self0, W_nbr0, b0,
           W_self1, W_nbr1, b1, W_self2, W_nbr2, b2):
    tbl_img = image_features.reshape(32, GRID * GRID * GRID).T

    src = edge_index[0]
    dst = edge_index[1]
    npad = EPAD - E
    ar = jnp.arange(npad, dtype=jnp.int32)
    src2 = jnp.concatenate([src, (ar * 37) % V]).reshape(EPAD // 128, 128)
    dst2 = jnp.concatenate([dst, V + ar % (VACC - V)]).reshape(EPAD // 128, 128)

    ws0p = jnp.pad(W_self0, ((0, 13), (0, 0)))
    wn0p = jnp.pad(W_nbr0, ((0, 13), (0, 0)))
    b0r = b0.reshape(1, 64)
    b1r = b1.reshape(1, 64)
    b2p = jnp.pad(b2, (0, 5)).reshape(1, 8)

    cidx, cw = _prep(vertices)
    cidx8 = jnp.pad(cidx.T, ((0, 0), (0, CPAD - V))).reshape(CIDX_N // 128, 128)
    corners = _corner_gather(tbl_img, cidx8).reshape(8, CPAD, 32)

    h0c = _build_h0(corners, cw, vertices)

    ep16 = _make_edge_pass(16)

    p0c = tuple(ep16(h, src2, dst2) for h in h0c)
    h1_0, h1_1, h1_2, h1_3, cnt = _dense0(h0c, p0c, ws0p, wn0p, b0r)

    h1c = (h1_0, h1_1, h1_2, h1_3)
    p1c = tuple(ep16(h, src2, dst2) for h in h1c)
    m2p, u2 = _dense1(h1c, p1c, cnt, W_self1, W_nbr1, b1r, W_self2, W_nbr2)

    p2 = ep16(m2p, src2, dst2)
    return _final(vertices, u2, m2p, p2, cnt, b2p)


# speedup vs baseline: 10.8589x; 1.1441x over previous
import functools

import jax
import jax.numpy as jnp
from jax import lax
from jax.experimental import pallas as pl
from jax.experimental.pallas import tpu as pltpu
from jax.experimental.pallas import tpu_sc as plsc

V = 50000
E = 800000
GRID = 64
NC = 2
NS = 16
NW = NC * NS

VACC = 51200
RPS = VACC // NS
EPAD = 819200
EBLK = EPAD // NW // 128
CBLK = 104
CIDX_N = NW * CBLK * 128
CPAD = CIDX_N // 8

_LEAK = 0.3
_SCALE = 0.1
_NBUF = 8


def _sc_mesh():
    return plsc.VectorSubcoreMesh(
        core_axis_name="c", subcore_axis_name="s", num_cores=NC, num_subcores=NS
    )


def _corner_gather(tbl, idx2):
    @functools.partial(
        pl.kernel,
        out_type=jax.ShapeDtypeStruct((CIDX_N, 32), jnp.float32),
        mesh=_sc_mesh(),
        compiler_params=pltpu.CompilerParams(use_tc_tiling_on_sc=False),
        scratch_types=[
            pltpu.VMEM((CBLK, 128), jnp.int32),
            pltpu.VMEM((_NBUF, 128, 32), jnp.float32),
        ] + [pltpu.SemaphoreType.DMA] * _NBUF,
    )
    def k(tbl_h, idx_h, out_h, idx_v, rows_v, *sems):
        c = lax.axis_index("c")
        s = lax.axis_index("s")
        wid = s * NC + c
        pltpu.sync_copy(idx_h.at[pl.ds(wid * CBLK, CBLK)], idx_v)

        for b in range(_NBUF):
            pltpu.async_copy(tbl_h.at[idx_v.at[b]], rows_v.at[b], sems[b])

        def group(g, _):
            for b in range(_NBUF):
                j = g * _NBUF + b
                pltpu.make_async_copy(
                    tbl_h.at[idx_v.at[j]], rows_v.at[b], sems[b]).wait()
                pltpu.sync_copy(
                    rows_v.at[b],
                    out_h.at[pl.ds(wid * CBLK * 128 + j * 128, 128)])
                jn = j + _NBUF

                @pl.when(jn < CBLK)
                def _():
                    pltpu.async_copy(
                        tbl_h.at[idx_v.at[jn]], rows_v.at[b], sems[b])
            return 0

        lax.fori_loop(0, CBLK // _NBUF, group, 0)

    return k(tbl, idx2)


def _make_edge_pass(D):
    @functools.partial(
        pl.kernel,
        out_type=jax.ShapeDtypeStruct((NC, VACC, D), jnp.float32),
        mesh=_sc_mesh(),
        compiler_params=pltpu.CompilerParams(use_tc_tiling_on_sc=False),
        scratch_types=[
            pltpu.VMEM((EBLK, 128), jnp.int32),
            pltpu.VMEM((EBLK, 128), jnp.int32),
            pltpu.VMEM((_NBUF, 128, D), jnp.float32),
            pltpu.VMEM_SHARED((VACC, D), jnp.float32),
        ] + [pltpu.SemaphoreType.DMA] * _NBUF,
    )
    def k(tbl_h, src_h, dst_h, out_h, src_v, dst_v, rows_v, acc, *sems):
        c = lax.axis_index("c")
        s = lax.axis_index("s")
        wid = s * NC + c

        def zrow(i, _):
            for kk in range(D // 16):
                rows_v[0, i, pl.ds(kk * 16, 16)] = jnp.zeros(
                    (16,), jnp.float32)
            return 0

        lax.fori_loop(0, 128, zrow, 0)

        def zacc(i, _):
            pltpu.sync_copy(
                rows_v.at[0], acc.at[pl.ds(s * RPS + i * 128, 128)])
            return 0

        lax.fori_loop(0, RPS // 128, zacc, 0)

        pltpu.sync_copy(src_h.at[pl.ds(wid * EBLK, EBLK)], src_v)
        pltpu.sync_copy(dst_h.at[pl.ds(wid * EBLK, EBLK)], dst_v)
        plsc.subcore_barrier()

        for b in range(_NBUF):
            pltpu.async_copy(tbl_h.at[src_v.at[b]], rows_v.at[b], sems[b])

        def group(g, _):
            for b in range(_NBUF):
                j = g * _NBUF + b
                pltpu.make_async_copy(
                    tbl_h.at[src_v.at[j]], rows_v.at[b], sems[b]).wait()
                pltpu.sync_copy(rows_v.at[b], acc.at[dst_v.at[j]], add=True)
                jn = j + _NBUF

                @pl.when(jn < EBLK)
                def _():
                    pltpu.async_copy(
                        tbl_h.at[src_v.at[jn]], rows_v.at[b], sems[b])
            return 0

        lax.fori_loop(0, EBLK // _NBUF, group, 0)
        plsc.subcore_barrier()
        pltpu.sync_copy(
            acc.at[pl.ds(s * RPS, RPS)], out_h.at[c, pl.ds(s * RPS, RPS)]
        )

    return k


def _prep(vt):
    def body(v_ref, cidx_ref, cw_ref):
        v = v_ref[...]
        vc = jnp.clip(v, 0.0, GRID - 1.0 - 1e-4)
        v0 = jnp.floor(vc)
        f = vc - v0
        i0 = v0.astype(jnp.int32)
        i1 = jnp.minimum(i0 + 1, GRID - 1)
        x0, y0, z0 = i0[0:1], i0[1:2], i0[2:3]
        x1, y1, z1 = i1[0:1], i1[1:2], i1[2:3]
        fx, fy, fz = f[0:1], f[1:2], f[2:3]
        gx, gy, gz = 1.0 - fx, 1.0 - fy, 1.0 - fz
        flat = lambda a, b, cc: (a * GRID + b) * GRID + cc
        cidx_ref[...] = jnp.concatenate(
            [flat(x0, y0, z0), flat(x1, y0, z0), flat(x0, y1, z0),
             flat(x0, y0, z1), flat(x1, y1, z0), flat(x1, y0, z1),
             flat(x0, y1, z1), flat(x1, y1, z1)], axis=0)
        cw_ref[...] = jnp.concatenate(
            [gx * gy * gz, fx * gy * gz, gx * fy * gz, gx * gy * fz,
             fx * fy * gz, fx * gy * fz, gx * fy * fz, fx * fy * fz], axis=0)

    return pl.pallas_call(
        body,
        grid=(VACC // 2048,),
        in_specs=[pl.BlockSpec((3, 2048), lambda i: (0, i))],
        out_specs=[
            pl.BlockSpec((8, 2048), lambda i: (0, i)),
            pl.BlockSpec((8, 2048), lambda i: (0, i)),
        ],
        out_shape=(
            jax.ShapeDtypeStruct((8, VACC), jnp.int32),
            jax.ShapeDtypeStruct((8, VACC), jnp.float32),
        ),
    )(vt)


_VB = 2000
_NBLK_TC = V // _VB


def _chunk_specs(n, width):
    return [pl.BlockSpec((_VB, width), lambda i: (i, 0)) for _ in range(n)]


def _part_specs(n):
    return [pl.BlockSpec((NC, _VB, 16), lambda i: (0, i, 0)) for _ in range(n)]


def _build_h0(corners, cw, vertices):
    def body(cor_ref, cw_ref, v_ref, h0_ref, h1_ref, h2_ref):
        cor = cor_ref[...]
        cw_b = cw_ref[...]
        feats = cor[0] * cw_b[:, 0:1]
        for cc in range(1, 8):
            feats = feats + cor[cc] * cw_b[:, cc:cc + 1]
        pos = v_ref[...] * (1.0 / (GRID - 1.0))
        nrow = feats.shape[0]
        h0_ref[...] = jnp.concatenate([pos, feats[:, :13]], axis=1)
        h1_ref[...] = feats[:, 13:29]
        h2_ref[...] = jnp.concatenate(
            [feats[:, 29:32],
             jnp.ones((nrow, 1), jnp.float32),
             jnp.zeros((nrow, 12), jnp.float32)], axis=1)

    return pl.pallas_call(
        body,
        grid=(_NBLK_TC,),
        in_specs=[
            pl.BlockSpec((8, _VB, 32), lambda i: (0, i, 0)),
            pl.BlockSpec((_VB, 8), lambda i: (i, 0)),
            pl.BlockSpec((_VB, 3), lambda i: (i, 0)),
        ],
        out_specs=_chunk_specs(3, 16),
        out_shape=tuple(
            jax.ShapeDtypeStruct((V, 16), jnp.float32) for _ in range(3)),
    )(corners, cw, vertices)


def _dense0(h0c, p0c, ws, wn, b):
    def body(ha_ref, hb_ref, hc_ref, pa_ref, pb_ref, pc_ref,
             ws_ref, wn_ref, b_ref, o0_ref, o1_ref, o2_ref, o3_ref, cnt_ref):
        h0 = jnp.concatenate(
            [ha_ref[...], hb_ref[...], hc_ref[...]], axis=1)
        s = jnp.concatenate(
            [pa_ref[0] + pa_ref[1], pb_ref[0] + pb_ref[1],
             pc_ref[0] + pc_ref[1]], axis=1)
        cr = s[:, 35:36]
        inv = 1.0 / jnp.maximum(cr, 1.0)
        r = cr * inv
        t = s * inv - r * h0
        z = (jnp.dot(h0, ws_ref[...], preferred_element_type=jnp.float32)
             + jnp.dot(t, wn_ref[...], preferred_element_type=jnp.float32)
             + b_ref[...])
        h1 = jnp.where(z >= 0, z, _LEAK * z)
        o0_ref[...] = h1[:, 0:16]
        o1_ref[...] = h1[:, 16:32]
        o2_ref[...] = h1[:, 32:48]
        o3_ref[...] = h1[:, 48:64]
        cnt_ref[...] = jnp.concatenate(
            [inv, r, jnp.zeros((inv.shape[0], 6), jnp.float32)], axis=1)

    return pl.pallas_call(
        body,
        grid=(_NBLK_TC,),
        in_specs=_chunk_specs(3, 16) + _part_specs(3) + [
            pl.BlockSpec((48, 64), lambda i: (0, 0)),
            pl.BlockSpec((48, 64), lambda i: (0, 0)),
            pl.BlockSpec((1, 64), lambda i: (0, 0)),
        ],
        out_specs=_chunk_specs(4, 16) + [
            pl.BlockSpec((_VB, 8), lambda i: (i, 0))],
        out_shape=tuple(
            jax.ShapeDtypeStruct((V, 16), jnp.float32) for _ in range(4)
        ) + (jax.ShapeDtypeStruct((V, 8), jnp.float32),),
    )(*h0c, *p0c, ws, wn, b)


def _dense1(h1c, p1c, cnt, ws1, wn1, b1, ws2, wn2):
    def body(h0_ref, h1_ref, h2_ref, h3_ref, p0_ref, p1_ref, p2_ref, p3_ref,
             cnt_ref, ws1_ref, wn1_ref, b1_ref, ws2_ref, wn2_ref,
             m_ref, u_ref):
        h1 = jnp.concatenate(
            [h0_ref[...], h1_ref[...], h2_ref[...], h3_ref[...]], axis=1)
        s = jnp.concatenate(
            [p0_ref[0] + p0_ref[1], p1_ref[0] + p1_ref[1],
             p2_ref[0] + p2_ref[1], p3_ref[0] + p3_ref[1]], axis=1)
        inv = cnt_ref[:, 0:1]
        r = cnt_ref[:, 1:2]
        t = s * inv - r * h1
        z = (jnp.dot(h1, ws1_ref[...], preferred_element_type=jnp.float32)
             + jnp.dot(t, wn1_ref[...], preferred_element_type=jnp.float32)
             + b1_ref[...])
        h2 = jnp.where(z >= 0, z, _LEAK * z)
        m2 = jnp.dot(h2, wn2_ref[...], preferred_element_type=jnp.float32)
        u2 = jnp.dot(h2, ws2_ref[...], preferred_element_type=jnp.float32)
        nrow = h2.shape[0]
        m_ref[...] = jnp.concatenate(
            [m2, jnp.zeros((nrow, 13), jnp.float32)], axis=1)
        u_ref[...] = jnp.concatenate(
            [u2, jnp.zeros((nrow, 5), jnp.float32)], axis=1)

    return pl.pallas_call(
        body,
        grid=(_NBLK_TC,),
        in_specs=_chunk_specs(4, 16) + _part_specs(4) + [
            pl.BlockSpec((_VB, 8), lambda i: (i, 0)),
            pl.BlockSpec((64, 64), lambda i: (0, 0)),
            pl.BlockSpec((64, 64), lambda i: (0, 0)),
            pl.BlockSpec((1, 64), lambda i: (0, 0)),
            pl.BlockSpec((64, 3), lambda i: (0, 0)),
            pl.BlockSpec((64, 3), lambda i: (0, 0)),
        ],
        out_specs=[
            pl.BlockSpec((_VB, 16), lambda i: (i, 0)),
            pl.BlockSpec((_VB, 8), lambda i: (i, 0)),
        ],
        out_shape=(
            jax.ShapeDtypeStruct((V, 16), jnp.float32),
            jax.ShapeDtypeStruct((V, 8), jnp.float32),
        ),
    )(*h1c, *p1c, cnt, ws1, wn1, b1, ws2, wn2)


def _final(vertices, u2, m2p, p2, cnt, b2p):
    def body(v_ref, u_ref, m_ref, p_ref, cnt_ref, b_ref, o_ref):
        s2 = p_ref[0] + p_ref[1]
        inv = cnt_ref[:, 0:1]
        r = cnt_ref[:, 1:2]
        agg = s2[:, 0:3] * inv - r * m_ref[:, 0:3]
        d = u_ref[:, 0:3] + agg + b_ref[:, 0:3]
        o_ref[...] = v_ref[...] + _SCALE * d

    return pl.pallas_call(
        body,
        grid=(_NBLK_TC,),
        in_specs=[
            pl.BlockSpec((_VB, 3), lambda i: (i, 0)),
            pl.BlockSpec((_VB, 8), lambda i: (i, 0)),
            pl.BlockSpec((_VB, 16), lambda i: (i, 0)),
            pl.BlockSpec((NC, _VB, 16), lambda i: (0, i, 0)),
            pl.BlockSpec((_VB, 8), lambda i: (i, 0)),
            pl.BlockSpec((1, 8), lambda i: (0, 0)),
        ],
        out_specs=pl.BlockSpec((_VB, 3), lambda i: (i, 0)),
        out_shape=jax.ShapeDtypeStruct((V, 3), jnp.float32),
    )(vertices, u2, m2p, p2, cnt, b2p)


def kernel(image_features, vertices, edge_index, W_self0, W_nbr0, b0,
           W_self1, W_nbr1, b1, W_self2, W_nbr2, b2):
    tbl_img = image_features.reshape(32, GRID * GRID * GRID).T

    src = edge_index[0]
    dst = edge_index[1]
    npad = EPAD - E
    ar = jnp.arange(npad, dtype=jnp.int32)
    src2 = jnp.concatenate([src, (ar * 37) % V]).reshape(EPAD // 128, 128)
    dst2 = jnp.concatenate([dst, V + ar % (VACC - V)]).reshape(EPAD // 128, 128)

    ws0p = jnp.pad(W_self0, ((0, 13), (0, 0)))
    wn0p = jnp.pad(W_nbr0, ((0, 13), (0, 0)))
    b0r = b0.reshape(1, 64)
    b1r = b1.reshape(1, 64)
    b2p = jnp.pad(b2, (0, 5)).reshape(1, 8)

    vtp = jnp.pad(vertices.T, ((0, 0), (0, VACC - V)))
    cidxt, cwt = _prep(vtp)
    cw = cwt.T
    cidx8 = jnp.pad(
        cidxt, ((0, 0), (0, CPAD - VACC))).reshape(CIDX_N // 128, 128)
    corners = _corner_gather(tbl_img, cidx8).reshape(8, CPAD, 32)

    h0c = _build_h0(corners, cw, vertices)

    ep16 = _make_edge_pass(16)

    p0c = tuple(ep16(h, src2, dst2) for h in h0c)
    h1_0, h1_1, h1_2, h1_3, cnt = _dense0(h0c, p0c, ws0p, wn0p, b0r)

    h1c = (h1_0, h1_1, h1_2, h1_3)
    p1c = tuple(ep16(h, src2, dst2) for h in h1c)
    m2p, u2 = _dense1(h1c, p1c, cnt, W_self1, W_nbr1, b1r, W_self2, W_nbr2)

    p2 = ep16(m2p, src2, dst2)
    return _final(vertices, u2, m2p, p2, cnt, b2p)


# speedup vs baseline: 10.9072x; 1.0045x over previous
import functools

import jax
import jax.numpy as jnp
from jax import lax
from jax.experimental import pallas as pl
from jax.experimental.pallas import tpu as pltpu
from jax.experimental.pallas import tpu_sc as plsc

V = 50000
E = 800000
GRID = 64
NC = 2
NS = 16
NW = NC * NS

VACC = 51200
RPS = VACC // NS
EPAD = 819200
EBLK = EPAD // NW // 128
CBLK = 104
CIDX_N = NW * CBLK * 128
CPAD = CIDX_N // 8

_LEAK = 0.3
_SCALE = 0.1
_NBUF = 8


def _sc_mesh():
    return plsc.VectorSubcoreMesh(
        core_axis_name="c", subcore_axis_name="s", num_cores=NC, num_subcores=NS
    )


def _corner_gather(tbl, idx2):
    @functools.partial(
        pl.kernel,
        out_type=jax.ShapeDtypeStruct((CIDX_N, 32), jnp.float32),
        mesh=_sc_mesh(),
        compiler_params=pltpu.CompilerParams(use_tc_tiling_on_sc=False),
        scratch_types=[
            pltpu.VMEM((CBLK, 128), jnp.int32),
            pltpu.VMEM((_NBUF, 128, 32), jnp.float32),
        ] + [pltpu.SemaphoreType.DMA] * _NBUF,
    )
    def k(tbl_h, idx_h, out_h, idx_v, rows_v, *sems):
        c = lax.axis_index("c")
        s = lax.axis_index("s")
        wid = s * NC + c
        pltpu.sync_copy(idx_h.at[pl.ds(wid * CBLK, CBLK)], idx_v)

        for b in range(_NBUF):
            pltpu.async_copy(tbl_h.at[idx_v.at[b]], rows_v.at[b], sems[b])

        def group(g, _):
            for b in range(_NBUF):
                j = g * _NBUF + b
                pltpu.make_async_copy(
                    tbl_h.at[idx_v.at[j]], rows_v.at[b], sems[b]).wait()
                pltpu.sync_copy(
                    rows_v.at[b],
                    out_h.at[pl.ds(wid * CBLK * 128 + j * 128, 128)])
                jn = j + _NBUF

                @pl.when(jn < CBLK)
                def _():
                    pltpu.async_copy(
                        tbl_h.at[idx_v.at[jn]], rows_v.at[b], sems[b])
            return 0

        lax.fori_loop(0, CBLK // _NBUF, group, 0)

    return k(tbl, idx2)


def _make_edge_pass(D):
    @functools.partial(
        pl.kernel,
        out_type=jax.ShapeDtypeStruct((NC, VACC, D), jnp.float32),
        mesh=_sc_mesh(),
        compiler_params=pltpu.CompilerParams(use_tc_tiling_on_sc=False),
        scratch_types=[
            pltpu.VMEM((EBLK, 128), jnp.int32),
            pltpu.VMEM((EBLK, 128), jnp.int32),
            pltpu.VMEM((_NBUF, 128, D), jnp.float32),
            pltpu.VMEM_SHARED((VACC, D), jnp.float32),
        ] + [pltpu.SemaphoreType.DMA] * _NBUF,
    )
    def k(tbl_h, src_h, dst_h, out_h, src_v, dst_v, rows_v, acc, *sems):
        c = lax.axis_index("c")
        s = lax.axis_index("s")
        wid = s * NC + c

        def zrow(i, _):
            for kk in range(D // 16):
                rows_v[0, i, pl.ds(kk * 16, 16)] = jnp.zeros(
                    (16,), jnp.float32)
            return 0

        lax.fori_loop(0, 128, zrow, 0)

        def zacc(i, _):
            pltpu.sync_copy(
                rows_v.at[0], acc.at[pl.ds(s * RPS + i * 128, 128)])
            return 0

        lax.fori_loop(0, RPS // 128, zacc, 0)

        pltpu.sync_copy(src_h.at[pl.ds(wid * EBLK, EBLK)], src_v)
        pltpu.sync_copy(dst_h.at[pl.ds(wid * EBLK, EBLK)], dst_v)
        plsc.subcore_barrier()

        for b in range(_NBUF):
            pltpu.async_copy(tbl_h.at[src_v.at[b]], rows_v.at[b], sems[b])

        def group(g, _):
            for b in range(_NBUF):
                j = g * _NBUF + b
                pltpu.make_async_copy(
                    tbl_h.at[src_v.at[j]], rows_v.at[b], sems[b]).wait()
                pltpu.sync_copy(rows_v.at[b], acc.at[dst_v.at[j]], add=True)
                jn = j + _NBUF

                @pl.when(jn < EBLK)
                def _():
                    pltpu.async_copy(
                        tbl_h.at[src_v.at[jn]], rows_v.at[b], sems[b])
            return 0

        lax.fori_loop(0, EBLK // _NBUF, group, 0)
        plsc.subcore_barrier()
        pltpu.sync_copy(
            acc.at[pl.ds(s * RPS, RPS)], out_h.at[c, pl.ds(s * RPS, RPS)]
        )

    return k


def _prep(vt):
    def body(v_ref, cidx_ref, cw_ref):
        v = v_ref[...]
        vc = jnp.clip(v, 0.0, GRID - 1.0 - 1e-4)
        v0 = jnp.floor(vc)
        f = vc - v0
        i0 = v0.astype(jnp.int32)
        i1 = jnp.minimum(i0 + 1, GRID - 1)
        x0, y0, z0 = i0[0:1], i0[1:2], i0[2:3]
        x1, y1, z1 = i1[0:1], i1[1:2], i1[2:3]
        fx, fy, fz = f[0:1], f[1:2], f[2:3]
        gx, gy, gz = 1.0 - fx, 1.0 - fy, 1.0 - fz
        flat = lambda a, b, cc: (a * GRID + b) * GRID + cc
        cidx_ref[...] = jnp.concatenate(
            [flat(x0, y0, z0), flat(x1, y0, z0), flat(x0, y1, z0),
             flat(x0, y0, z1), flat(x1, y1, z0), flat(x1, y0, z1),
             flat(x0, y1, z1), flat(x1, y1, z1)], axis=0)
        cw_ref[...] = jnp.concatenate(
            [gx * gy * gz, fx * gy * gz, gx * fy * gz, gx * gy * fz,
             fx * fy * gz, fx * gy * fz, gx * fy * fz, fx * fy * fz], axis=0)

    return pl.pallas_call(
        body,
        grid=(VACC // 2048,),
        in_specs=[pl.BlockSpec((3, 2048), lambda i: (0, i))],
        out_specs=[
            pl.BlockSpec((8, 2048), lambda i: (0, i)),
            pl.BlockSpec((8, 2048), lambda i: (0, i)),
        ],
        out_shape=(
            jax.ShapeDtypeStruct((8, VACC), jnp.int32),
            jax.ShapeDtypeStruct((8, VACC), jnp.float32),
        ),
    )(vt)


_VB = 2000
_NBLK_TC = V // _VB


def _chunk_specs(n, width):
    return [pl.BlockSpec((_VB, width), lambda i: (i, 0)) for _ in range(n)]


def _part_specs(n):
    return [pl.BlockSpec((NC, _VB, 16), lambda i: (0, i, 0)) for _ in range(n)]


def _build_h0(corners, cw, vertices):
    def body(cor_ref, cw_ref, v_ref, h0_ref, h1_ref, h2_ref):
        cor = cor_ref[...]
        cw_b = cw_ref[...]
        feats = cor[0] * cw_b[:, 0:1]
        for cc in range(1, 8):
            feats = feats + cor[cc] * cw_b[:, cc:cc + 1]
        pos = v_ref[...] * (1.0 / (GRID - 1.0))
        nrow = feats.shape[0]
        h0_ref[...] = jnp.concatenate([pos, feats[:, :13]], axis=1)
        h1_ref[...] = feats[:, 13:29]
        h2_ref[...] = jnp.concatenate(
            [feats[:, 29:32],
             jnp.ones((nrow, 1), jnp.float32),
             jnp.zeros((nrow, 12), jnp.float32)], axis=1)

    return pl.pallas_call(
        body,
        grid=(_NBLK_TC,),
        in_specs=[
            pl.BlockSpec((8, _VB, 32), lambda i: (0, i, 0)),
            pl.BlockSpec((_VB, 8), lambda i: (i, 0)),
            pl.BlockSpec((_VB, 3), lambda i: (i, 0)),
        ],
        out_specs=_chunk_specs(3, 16),
        out_shape=tuple(
            jax.ShapeDtypeStruct((V, 16), jnp.float32) for _ in range(3)),
    )(corners, cw, vertices)


def _dense0(h0c, p0c, ws, wn, b):
    def body(ha_ref, hb_ref, hc_ref, pa_ref, pb_ref, pc_ref,
             ws_ref, wn_ref, b_ref, o0_ref, o1_ref, o2_ref, o3_ref, cnt_ref):
        hs = (ha_ref[...], hb_ref[...], hc_ref[...])
        ss = (pa_ref[0] + pa_ref[1], pb_ref[0] + pb_ref[1],
              pc_ref[0] + pc_ref[1])
        cr = ss[2][:, 3:4]
        inv = 1.0 / jnp.maximum(cr, 1.0)
        r = cr * inv
        z = b_ref[...]
        for kk in range(3):
            t = ss[kk] * inv - r * hs[kk]
            z = z + jnp.dot(hs[kk], ws_ref[16 * kk:16 * kk + 16, :],
                            preferred_element_type=jnp.float32)
            z = z + jnp.dot(t, wn_ref[16 * kk:16 * kk + 16, :],
                            preferred_element_type=jnp.float32)
        h1 = jnp.where(z >= 0, z, _LEAK * z)
        o0_ref[...] = h1[:, 0:16]
        o1_ref[...] = h1[:, 16:32]
        o2_ref[...] = h1[:, 32:48]
        o3_ref[...] = h1[:, 48:64]
        cnt_ref[...] = jnp.concatenate(
            [inv, r, jnp.zeros((inv.shape[0], 6), jnp.float32)], axis=1)

    return pl.pallas_call(
        body,
        grid=(_NBLK_TC,),
        in_specs=_chunk_specs(3, 16) + _part_specs(3) + [
            pl.BlockSpec((48, 64), lambda i: (0, 0)),
            pl.BlockSpec((48, 64), lambda i: (0, 0)),
            pl.BlockSpec((1, 64), lambda i: (0, 0)),
        ],
        out_specs=_chunk_specs(4, 16) + [
            pl.BlockSpec((_VB, 8), lambda i: (i, 0))],
        out_shape=tuple(
            jax.ShapeDtypeStruct((V, 16), jnp.float32) for _ in range(4)
        ) + (jax.ShapeDtypeStruct((V, 8), jnp.float32),),
    )(*h0c, *p0c, ws, wn, b)


def _dense1(h1c, p1c, cnt, ws1, wn1, b1, ws2, wn2):
    def body(h0_ref, h1_ref, h2_ref, h3_ref, p0_ref, p1_ref, p2_ref, p3_ref,
             cnt_ref, ws1_ref, wn1_ref, b1_ref, ws2_ref, wn2_ref,
             m_ref, u_ref):
        hs = (h0_ref[...], h1_ref[...], h2_ref[...], h3_ref[...])
        ss = (p0_ref[0] + p0_ref[1], p1_ref[0] + p1_ref[1],
              p2_ref[0] + p2_ref[1], p3_ref[0] + p3_ref[1])
        inv = cnt_ref[:, 0:1]
        r = cnt_ref[:, 1:2]
        z = b1_ref[...]
        for kk in range(4):
            t = ss[kk] * inv - r * hs[kk]
            z = z + jnp.dot(hs[kk], ws1_ref[16 * kk:16 * kk + 16, :],
                            preferred_element_type=jnp.float32)
            z = z + jnp.dot(t, wn1_ref[16 * kk:16 * kk + 16, :],
                            preferred_element_type=jnp.float32)
        h2 = jnp.where(z >= 0, z, _LEAK * z)
        m2 = jnp.dot(h2, wn2_ref[...], preferred_element_type=jnp.float32)
        u2 = jnp.dot(h2, ws2_ref[...], preferred_element_type=jnp.float32)
        nrow = h2.shape[0]
        m_ref[...] = jnp.concatenate(
            [m2, jnp.zeros((nrow, 13), jnp.float32)], axis=1)
        u_ref[...] = jnp.concatenate(
            [u2, jnp.zeros((nrow, 5), jnp.float32)], axis=1)

    return pl.pallas_call(
        body,
        grid=(_NBLK_TC,),
        in_specs=_chunk_specs(4, 16) + _part_specs(4) + [
            pl.BlockSpec((_VB, 8), lambda i: (i, 0)),
            pl.BlockSpec((64, 64), lambda i: (0, 0)),
            pl.BlockSpec((64, 64), lambda i: (0, 0)),
            pl.BlockSpec((1, 64), lambda i: (0, 0)),
            pl.BlockSpec((64, 3), lambda i: (0, 0)),
            pl.BlockSpec((64, 3), lambda i: (0, 0)),
        ],
        out_specs=[
            pl.BlockSpec((_VB, 16), lambda i: (i, 0)),
            pl.BlockSpec((_VB, 8), lambda i: (i, 0)),
        ],
        out_shape=(
            jax.ShapeDtypeStruct((V, 16), jnp.float32),
            jax.ShapeDtypeStruct((V, 8), jnp.float32),
        ),
    )(*h1c, *p1c, cnt, ws1, wn1, b1, ws2, wn2)


def _final(vertices, u2, m2p, p2, cnt, b2p):
    def body(v_ref, u_ref, m_ref, p_ref, cnt_ref, b_ref, o_ref):
        s2 = p_ref[0] + p_ref[1]
        inv = cnt_ref[:, 0:1]
        r = cnt_ref[:, 1:2]
        agg = s2[:, 0:3] * inv - r * m_ref[:, 0:3]
        d = u_ref[:, 0:3] + agg + b_ref[:, 0:3]
        o_ref[...] = v_ref[...] + _SCALE * d

    return pl.pallas_call(
        body,
        grid=(_NBLK_TC,),
        in_specs=[
            pl.BlockSpec((_VB, 3), lambda i: (i, 0)),
            pl.BlockSpec((_VB, 8), lambda i: (i, 0)),
            pl.BlockSpec((_VB, 16), lambda i: (i, 0)),
            pl.BlockSpec((NC, _VB, 16), lambda i: (0, i, 0)),
            pl.BlockSpec((_VB, 8), lambda i: (i, 0)),
            pl.BlockSpec((1, 8), lambda i: (0, 0)),
        ],
        out_specs=pl.BlockSpec((_VB, 3), lambda i: (i, 0)),
        out_shape=jax.ShapeDtypeStruct((V, 3), jnp.float32),
    )(vertices, u2, m2p, p2, cnt, b2p)


def kernel(image_features, vertices, edge_index, W_self0, W_nbr0, b0,
           W_self1, W_nbr1, b1, W_self2, W_nbr2, b2):
    tbl_img = image_features.reshape(32, GRID * GRID * GRID).T

    src = edge_index[0]
    dst = edge_index[1]
    npad = EPAD - E
    ar = jnp.arange(npad, dtype=jnp.int32)
    src2 = jnp.concatenate([src, (ar * 37) % V]).reshape(EPAD // 128, 128)
    dst2 = jnp.concatenate([dst, V + ar % (VACC - V)]).reshape(EPAD // 128, 128)

    ws0p = jnp.pad(W_self0, ((0, 13), (0, 0)))
    wn0p = jnp.pad(W_nbr0, ((0, 13), (0, 0)))
    b0r = b0.reshape(1, 64)
    b1r = b1.reshape(1, 64)
    b2p = jnp.pad(b2, (0, 5)).reshape(1, 8)

    vtp = jnp.pad(vertices.T, ((0, 0), (0, VACC - V)))
    cidxt, cwt = _prep(vtp)
    cw = cwt.T
    cidx8 = jnp.pad(
        cidxt, ((0, 0), (0, CPAD - VACC))).reshape(CIDX_N // 128, 128)
    corners = _corner_gather(tbl_img, cidx8).reshape(8, CPAD, 32)

    h0c = _build_h0(corners, cw, vertices)

    ep16 = _make_edge_pass(16)

    p0c = tuple(ep16(h, src2, dst2) for h in h0c)
    h1_0, h1_1, h1_2, h1_3, cnt = _dense0(h0c, p0c, ws0p, wn0p, b0r)

    h1c = (h1_0, h1_1, h1_2, h1_3)
    p1c = tuple(ep16(h, src2, dst2) for h in h1c)
    m2p, u2 = _dense1(h1c, p1c, cnt, W_self1, W_nbr1, b1r, W_self2, W_nbr2)

    p2 = ep16(m2p, src2, dst2)
    return _final(vertices, u2, m2p, p2, cnt, b2p)
